# baseline (speedup 1.0000x reference)
import jax
import jax.numpy as jnp
from jax import lax
from jax.experimental import pallas as pl
from jax.experimental.pallas import tpu as pltpu

N = 17
D = 768
BS = 4096
B = 8
BN = B * N


def _pfd_block(m_ref, m1_ref, out_ref):
    m2 = m_ref[...].reshape(BN, D)
    pwf2 = m2 * m1_ref[...].reshape(BN, D)

    dots = lax.dot_general(
        m2, pwf2, (((1,), (1,)), ((), ())), preferred_element_type=jnp.float32
    )

    na_col = jnp.sqrt(jnp.sum(m2 * m2, axis=1, keepdims=True))
    sq = pwf2 * pwf2
    nb_row = jnp.sqrt(
        lax.dot_general(
            jnp.ones((1, D), jnp.float32), sq, (((1,), (1,)), ((), ())),
            preferred_element_type=jnp.float32,
        )
    )
    denom = jnp.maximum(na_col * nb_row, 1e-8)
    sim = dots / denom

    r_blk = lax.broadcasted_iota(jnp.int32, (BN, BN), 0) // N
    c_idx = lax.broadcasted_iota(jnp.int32, (BN, BN), 1)
    valid = r_blk == (c_idx // N)
    simm = jnp.where(valid, sim, -jnp.inf)

    amax = jnp.argmax(simm, axis=1).astype(jnp.int32)
    oh = (amax[:, None] == c_idx).astype(jnp.float32)

    gathered = lax.dot_general(
        oh, pwf2, (((1,), (0,)), ((), ())), preferred_element_type=jnp.float32
    )
    out_ref[...] = (m2 + gathered).reshape(B, N, D)


@jax.jit
def kernel(matrix, matrix1):
    return pl.pallas_call(
        _pfd_block,
        grid=(BS // B,),
        in_specs=[
            pl.BlockSpec((B, N, D), lambda i: (i, 0, 0)),
            pl.BlockSpec((B, N, D), lambda i: (i, 0, 0)),
        ],
        out_specs=pl.BlockSpec((B, N, D), lambda i: (i, 0, 0)),
        out_shape=jax.ShapeDtypeStruct((BS, N, D), jnp.float32),
        compiler_params=pltpu.CompilerParams(
            dimension_semantics=("parallel",),
        ),
    )(matrix, matrix1)


# speedup vs baseline: 1.0796x; 1.0796x over previous
import jax
import jax.numpy as jnp
from jax import lax
from jax.experimental import pallas as pl
from jax.experimental.pallas import tpu as pltpu

N = 17
D = 768
BS = 4096
B = 8


def _pfd_block(m_ref, m1_ref, out_ref):
    row_j = lax.broadcasted_iota(jnp.int32, (N, N), 0)
    for b in range(B):
        m = m_ref[b]
        pwf = m * m1_ref[b]

        dotsT = lax.dot_general(
            pwf, m, (((1,), (1,)), ((), ())),
            preferred_element_type=jnp.float32,
        )
        nb = jnp.sqrt(jnp.sum(pwf * pwf, axis=1, keepdims=True))
        simT = dotsT / jnp.maximum(nb, 1e-30)

        ind = jnp.argmax(simT, axis=0).astype(jnp.int32)
        ohT = (ind[None, :] == row_j).astype(jnp.float32)

        gathered = lax.dot_general(
            ohT, pwf, (((0,), (0,)), ((), ())),
            preferred_element_type=jnp.float32,
        )
        out_ref[b] = m + gathered


@jax.jit
def kernel(matrix, matrix1):
    return pl.pallas_call(
        _pfd_block,
        grid=(BS // B,),
        in_specs=[
            pl.BlockSpec((B, N, D), lambda i: (i, 0, 0)),
            pl.BlockSpec((B, N, D), lambda i: (i, 0, 0)),
        ],
        out_specs=pl.BlockSpec((B, N, D), lambda i: (i, 0, 0)),
        out_shape=jax.ShapeDtypeStruct((BS, N, D), jnp.float32),
        compiler_params=pltpu.CompilerParams(
            dimension_semantics=("parallel",),
        ),
    )(matrix, matrix1)


# speedup vs baseline: 2.7088x; 2.5090x over previous
import jax
import jax.numpy as jnp
from jax import lax
from jax.experimental import pallas as pl
from jax.experimental.pallas import tpu as pltpu

N = 17
D = 768
BS = 4096
B = 8
BN = N * B


def _pfd_block(m_ref, m1_ref, out_ref):
    m2 = m_ref[...].reshape(BN, D)
    pwf2 = m2 * m1_ref[...].reshape(BN, D)

    dotsT = lax.dot_general(
        pwf2, m2, (((1,), (1,)), ((), ())),
        preferred_element_type=jnp.float32,
    )
    nb = jnp.sqrt(jnp.sum(pwf2 * pwf2, axis=1, keepdims=True))
    simT = dotsT / jnp.maximum(nb, 1e-30)

    r_idx = lax.broadcasted_iota(jnp.int32, (BN, BN), 0)
    c_idx = lax.broadcasted_iota(jnp.int32, (BN, BN), 1)
    valid = (r_idx % B) == (c_idx % B)
    simm = jnp.where(valid, simT, -jnp.inf)

    rmax = jnp.argmax(simm, axis=0).astype(jnp.int32)
    ohT = (rmax[None, :] == r_idx).astype(jnp.float32)

    gathered = lax.dot_general(
        ohT, pwf2, (((0,), (0,)), ((), ())),
        preferred_element_type=jnp.float32,
    )
    out_ref[...] = (m2 + gathered).reshape(N, B, D)


@jax.jit
def kernel(matrix, matrix1):
    mt = jnp.transpose(matrix, (1, 0, 2))
    m1t = jnp.transpose(matrix1, (1, 0, 2))
    out_t = pl.pallas_call(
        _pfd_block,
        grid=(BS // B,),
        in_specs=[
            pl.BlockSpec((N, B, D), lambda i: (0, i, 0)),
            pl.BlockSpec((N, B, D), lambda i: (0, i, 0)),
        ],
        out_specs=pl.BlockSpec((N, B, D), lambda i: (0, i, 0)),
        out_shape=jax.ShapeDtypeStruct((N, BS, D), jnp.float32),
        compiler_params=pltpu.CompilerParams(
            dimension_semantics=("parallel",),
        ),
    )(mt, m1t)
    return jnp.transpose(out_t, (1, 0, 2))


# speedup vs baseline: 5.2380x; 1.9337x over previous
import jax
import jax.numpy as jnp
from jax import lax
from jax.experimental import pallas as pl
from jax.experimental.pallas import tpu as pltpu

N = 17
D = 768
BS = 4096
SB = 8
NSUB = 4
B = SB * NSUB
BN = N * SB


def _pfd_sub(m2, pwf2, r_idx, c_idx, valid):
    dotsT = lax.dot_general(
        pwf2, m2, (((1,), (1,)), ((), ())),
        preferred_element_type=jnp.float32,
    )
    nb = jnp.sqrt(jnp.sum(pwf2 * pwf2, axis=1, keepdims=True))
    simT = dotsT / jnp.maximum(nb, 1e-30)

    simm = jnp.where(valid, simT, -jnp.inf)
    rmax = jnp.argmax(simm, axis=0).astype(jnp.int32)
    ohT = (rmax[None, :] == r_idx).astype(jnp.float32)

    gathered = lax.dot_general(
        ohT, pwf2, (((0,), (0,)), ((), ())),
        preferred_element_type=jnp.float32,
    )
    return m2 + gathered


def _pfd_block(m_ref, m1_ref, out_ref):
    r_idx = lax.broadcasted_iota(jnp.int32, (BN, BN), 0)
    c_idx = lax.broadcasted_iota(jnp.int32, (BN, BN), 1)
    valid = (r_idx % SB) == (c_idx % SB)
    for s in range(NSUB):
        m2 = m_ref[:, s * SB:(s + 1) * SB, :].reshape(BN, D)
        pwf2 = m2 * m1_ref[:, s * SB:(s + 1) * SB, :].reshape(BN, D)
        out_ref[:, s * SB:(s + 1) * SB, :] = _pfd_sub(
            m2, pwf2, r_idx, c_idx, valid
        ).reshape(N, SB, D)


@jax.jit
def kernel(matrix, matrix1):
    mt = jnp.transpose(matrix, (1, 0, 2))
    m1t = jnp.transpose(matrix1, (1, 0, 2))
    out_t = pl.pallas_call(
        _pfd_block,
        grid=(BS // B,),
        in_specs=[
            pl.BlockSpec((N, B, D), lambda i: (0, i, 0)),
            pl.BlockSpec((N, B, D), lambda i: (0, i, 0)),
        ],
        out_specs=pl.BlockSpec((N, B, D), lambda i: (0, i, 0)),
        out_shape=jax.ShapeDtypeStruct((N, BS, D), jnp.float32),
        compiler_params=pltpu.CompilerParams(
            dimension_semantics=("parallel",),
        ),
    )(mt, m1t)
    return jnp.transpose(out_t, (1, 0, 2))


# speedup vs baseline: 6.4715x; 1.2355x over previous
import jax
import jax.numpy as jnp
from jax import lax
from jax.experimental import pallas as pl
from jax.experimental.pallas import tpu as pltpu

N = 17
D = 768
BS = 4096
SB = 8
NSUB = 8
B = SB * NSUB
BN = N * SB


def _pfd_sub(m2, pwf2, r_idx, c_idx, valid):
    dotsT = lax.dot_general(
        pwf2, m2, (((1,), (1,)), ((), ())),
        preferred_element_type=jnp.float32,
    )
    nb = jnp.sqrt(jnp.sum(pwf2 * pwf2, axis=1, keepdims=True))
    simT = dotsT / jnp.maximum(nb, 1e-30)

    simm = jnp.where(valid, simT, -jnp.inf)
    rmax = jnp.argmax(simm, axis=0).astype(jnp.int32)
    ohT = (rmax[None, :] == r_idx).astype(jnp.float32)

    gathered = lax.dot_general(
        ohT, pwf2, (((0,), (0,)), ((), ())),
        preferred_element_type=jnp.float32,
    )
    return m2 + gathered


def _pfd_block(m_ref, m1_ref, out_ref):
    r_idx = lax.broadcasted_iota(jnp.int32, (BN, BN), 0)
    c_idx = lax.broadcasted_iota(jnp.int32, (BN, BN), 1)
    valid = (r_idx % SB) == (c_idx % SB)
    for s in range(NSUB):
        m2 = m_ref[:, s * SB:(s + 1) * SB, :].reshape(BN, D)
        pwf2 = m2 * m1_ref[:, s * SB:(s + 1) * SB, :].reshape(BN, D)
        out_ref[:, s * SB:(s + 1) * SB, :] = _pfd_sub(
            m2, pwf2, r_idx, c_idx, valid
        ).reshape(N, SB, D)


@jax.jit
def kernel(matrix, matrix1):
    mt = jnp.transpose(matrix, (1, 0, 2))
    m1t = jnp.transpose(matrix1, (1, 0, 2))
    out_t = pl.pallas_call(
        _pfd_block,
        grid=(BS // B,),
        in_specs=[
            pl.BlockSpec((N, B, D), lambda i: (0, i, 0)),
            pl.BlockSpec((N, B, D), lambda i: (0, i, 0)),
        ],
        out_specs=pl.BlockSpec((N, B, D), lambda i: (0, i, 0)),
        out_shape=jax.ShapeDtypeStruct((N, BS, D), jnp.float32),
        compiler_params=pltpu.CompilerParams(
            dimension_semantics=("parallel",),
        ),
    )(mt, m1t)
    return jnp.transpose(out_t, (1, 0, 2))


# speedup vs baseline: 6.5070x; 1.0055x over previous
import jax
import jax.numpy as jnp
from jax import lax
from jax.experimental import pallas as pl
from jax.experimental.pallas import tpu as pltpu

N = 17
D = 768
BS = 4096
SB = 8
NSUB = 16
B = SB * NSUB
BN = N * SB


def _pfd_sub(m2, pwf2, r_idx, c_idx, valid):
    dotsT = lax.dot_general(
        pwf2, m2, (((1,), (1,)), ((), ())),
        preferred_element_type=jnp.float32,
    )
    nb = jnp.sqrt(jnp.sum(pwf2 * pwf2, axis=1, keepdims=True))
    simT = dotsT / jnp.maximum(nb, 1e-30)

    simm = jnp.where(valid, simT, -jnp.inf)
    rmax = jnp.argmax(simm, axis=0).astype(jnp.int32)
    ohT = (rmax[None, :] == r_idx).astype(jnp.float32)

    gathered = lax.dot_general(
        ohT, pwf2, (((0,), (0,)), ((), ())),
        preferred_element_type=jnp.float32,
    )
    return m2 + gathered


def _pfd_block(m_ref, m1_ref, out_ref):
    r_idx = lax.broadcasted_iota(jnp.int32, (BN, BN), 0)
    c_idx = lax.broadcasted_iota(jnp.int32, (BN, BN), 1)
    valid = (r_idx % SB) == (c_idx % SB)
    for s in range(NSUB):
        m2 = m_ref[:, s * SB:(s + 1) * SB, :].reshape(BN, D)
        pwf2 = m2 * m1_ref[:, s * SB:(s + 1) * SB, :].reshape(BN, D)
        out_ref[:, s * SB:(s + 1) * SB, :] = _pfd_sub(
            m2, pwf2, r_idx, c_idx, valid
        ).reshape(N, SB, D)


@jax.jit
def kernel(matrix, matrix1):
    mt = jnp.transpose(matrix, (1, 0, 2))
    m1t = jnp.transpose(matrix1, (1, 0, 2))
    out_t = pl.pallas_call(
        _pfd_block,
        grid=(BS // B,),
        in_specs=[
            pl.BlockSpec((N, B, D), lambda i: (0, i, 0)),
            pl.BlockSpec((N, B, D), lambda i: (0, i, 0)),
        ],
        out_specs=pl.BlockSpec((N, B, D), lambda i: (0, i, 0)),
        out_shape=jax.ShapeDtypeStruct((N, BS, D), jnp.float32),
        compiler_params=pltpu.CompilerParams(
            dimension_semantics=("parallel",),
        ),
    )(mt, m1t)
    return jnp.transpose(out_t, (1, 0, 2))
